# baseline (speedup 1.0000x reference)
import jax
import jax.numpy as jnp
from jax import lax
from jax.experimental import pallas as pl
from jax.experimental.pallas import tpu as pltpu

_NSPLIT = 8


def _make_tc_kernel(Lx, Nx, M):
    D = Lx - 2
    NB = Nx // _NSPLIT

    def chunk_copy(x_hbm, buf, sems, tt, slot, j):
        return pltpu.make_async_copy(
            x_hbm.at[tt + 1, pl.ds(j * NB, NB), :],
            buf.at[slot, pl.ds(j * NB, NB), :],
            sems.at[slot, j],
        )

    def body(seq_ref, x_hbm, out_ref, buf, acc_ref, sems):
        t = pl.program_id(0)
        slot = lax.rem(t, 2)

        @pl.when(t == 0)
        def _prime():
            acc_ref[0] = jnp.float32(0)
            acc_ref[1] = jnp.float32(0)
            for j in range(_NSPLIT):
                chunk_copy(x_hbm, buf, sems, 0, 0, j).start()

        @pl.when(t + 1 < D)
        def _issue_next():
            nslot = lax.rem(t + 1, 2)
            for j in range(_NSPLIT):
                chunk_copy(x_hbm, buf, sems, t + 1, nslot, j).start()

        for j in range(_NSPLIT):
            chunk_copy(x_hbm, buf, sems, t, slot, j).wait()

        seq = seq_ref[...]
        nz = jnp.minimum(seq, 1)
        r = lax.broadcasted_iota(jnp.int32, (D, D), 0)
        c = lax.broadcasted_iota(jnp.int32, (D, D), 1)
        tri = (r >= c).astype(jnp.float32)
        zcount = jnp.dot(tri, (1 - nz).astype(jnp.float32),
                         preferred_element_type=jnp.float32)
        first0 = (1 - nz) * jnp.where(zcount == 1.0, 1, 0)
        eff = seq + first0 * (M - 1)
        rowsel = lax.broadcasted_iota(jnp.int32, (D, Nx), 0) == t
        eff_t = jnp.sum(jnp.where(rowsel, eff, 0), axis=0)
        msk_t = jnp.minimum(eff_t, 1).astype(jnp.float32)

        x = buf[pl.ds(slot, 1), :, :][0]
        lane = lax.broadcasted_iota(jnp.int32, (Nx, M), 1)
        sel = lane == eff_t.reshape(Nx, 1)
        vals = jnp.sum(jnp.where(sel, x, jnp.float32(0)), axis=1)
        acc_ref[0] += jnp.sum(vals * msk_t)
        acc_ref[1] += jnp.sum(msk_t)

        @pl.when(t == D - 1)
        def _fin():
            out_ref[...] = jnp.full((1, 1), -(acc_ref[0] / acc_ref[1]),
                                    jnp.float32)

    return pl.pallas_call(
        body,
        grid=(D,),
        in_specs=[
            pl.BlockSpec((D, Nx), lambda t: (0, 0)),
            pl.BlockSpec(memory_space=pltpu.MemorySpace.HBM),
        ],
        out_specs=pl.BlockSpec((1, 1), lambda t: (0, 0)),
        out_shape=jax.ShapeDtypeStruct((1, 1), jnp.float32),
        scratch_shapes=[
            pltpu.VMEM((2, Nx, M), jnp.float32),
            pltpu.SMEM((2,), jnp.float32),
            pltpu.SemaphoreType.DMA((2, _NSPLIT)),
        ],
    )


def kernel(input, seq):
    Lx, Nx, M = input.shape
    out = _make_tc_kernel(Lx, Nx, M)(seq, input)
    return out[0, 0]
